# Initial kernel scaffold; baseline (speedup 1.0000x reference)
import jax
import jax.numpy as jnp
from jax.experimental import pallas as pl


def kernel(input_ids, embedding, W, b):
    raise NotImplementedError("write your pallas kernel here")


# speedup vs baseline: 3.3681x; 3.3681x over previous
import functools

import jax
import jax.numpy as jnp
from jax import lax
from jax.experimental import pallas as pl
from jax.experimental.pallas import tpu as pltpu
from jax.experimental.pallas import tpu_sc as plsc

_NUM_CORES = 2
_NUM_SUBCORES = 16
_NUM_WORKERS = _NUM_CORES * _NUM_SUBCORES


def _transform_body(emb_ref, w_ref, b_ref, out_ref):
    out_ref[...] = (
        lax.dot_general(
            emb_ref[...], w_ref[...],
            (((1,), (1,)), ((), ())),
            preferred_element_type=jnp.float32,
        )
        + b_ref[...]
    )


def _transform_table(embedding, W, b):
    vocab, hidden = embedding.shape
    return pl.pallas_call(
        _transform_body,
        out_shape=jax.ShapeDtypeStruct((vocab, hidden), jnp.float32),
    )(embedding, W, b.reshape(1, hidden))


@functools.lru_cache(maxsize=None)
def _make_gather(vocab, hidden, num_tokens, chunk):
    b_per_w = num_tokens // _NUM_WORKERS
    n_chunks = b_per_w // chunk
    assert b_per_w % chunk == 0 and num_tokens % _NUM_WORKERS == 0
    mesh = plsc.VectorSubcoreMesh(
        core_axis_name="c", subcore_axis_name="s",
        num_cores=_NUM_CORES, num_subcores=_NUM_SUBCORES,
    )

    @functools.partial(
        pl.kernel,
        out_type=jax.ShapeDtypeStruct((num_tokens, hidden), jnp.float32),
        mesh=mesh,
        scratch_types=[
            pltpu.VMEM((chunk,), jnp.int32),
            pltpu.VMEM((chunk, hidden), jnp.float32),
            pltpu.SemaphoreType.DMA,
        ],
        compiler_params=pltpu.CompilerParams(use_tc_tiling_on_sc=False),
    )
    def gather_kernel(table_hbm, idx_hbm, out_hbm, idx_v, rows_v, sem):
        wid = lax.axis_index("s") * _NUM_CORES + lax.axis_index("c")
        base = wid * b_per_w

        def body(i, carry):
            b0 = base + i * chunk
            pltpu.sync_copy(idx_hbm.at[pl.ds(b0, chunk)], idx_v)
            pltpu.async_copy(table_hbm.at[idx_v], rows_v, sem).wait()
            pltpu.sync_copy(rows_v, out_hbm.at[pl.ds(b0, chunk)])
            return carry

        lax.fori_loop(0, n_chunks, body, 0)

    return gather_kernel


def kernel(input_ids, embedding, W, b):
    batch, seq = input_ids.shape
    vocab, hidden = embedding.shape
    num_tokens = batch * seq
    table = _transform_table(embedding, W, b)
    flat_ids = input_ids.reshape(num_tokens).astype(jnp.int32)
    out = _make_gather(vocab, hidden, num_tokens, 512)(table, flat_ids)
    return out.reshape(batch, seq, hidden)


# speedup vs baseline: 3.3688x; 1.0002x over previous
import functools

import jax
import jax.numpy as jnp
from jax import lax
from jax.experimental import pallas as pl
from jax.experimental.pallas import tpu as pltpu
from jax.experimental.pallas import tpu_sc as plsc

_NUM_CORES = 2
_NUM_SUBCORES = 16
_NUM_WORKERS = _NUM_CORES * _NUM_SUBCORES


def _transform_body(emb_ref, w_ref, b_ref, out_ref):
    out_ref[...] = (
        lax.dot_general(
            emb_ref[...], w_ref[...],
            (((1,), (1,)), ((), ())),
            preferred_element_type=jnp.float32,
        )
        + b_ref[...]
    )


def _transform_table(embedding, W, b):
    vocab, hidden = embedding.shape
    return pl.pallas_call(
        _transform_body,
        out_shape=jax.ShapeDtypeStruct((vocab, hidden), jnp.float32),
    )(embedding, W, b.reshape(1, hidden))


@functools.lru_cache(maxsize=None)
def _make_gather(vocab, hidden, num_tokens, chunk):
    b_per_w = num_tokens // _NUM_WORKERS
    n_chunks = b_per_w // chunk
    n_outer = n_chunks // 2
    assert b_per_w % chunk == 0 and num_tokens % _NUM_WORKERS == 0
    assert n_chunks % 2 == 0
    mesh = plsc.VectorSubcoreMesh(
        core_axis_name="c", subcore_axis_name="s",
        num_cores=_NUM_CORES, num_subcores=_NUM_SUBCORES,
    )

    @functools.partial(
        pl.kernel,
        out_type=jax.ShapeDtypeStruct((num_tokens, hidden), jnp.float32),
        mesh=mesh,
        scratch_types=[
            pltpu.VMEM((2, chunk), jnp.int32),
            pltpu.VMEM((2, chunk, hidden), jnp.float32),
            pltpu.SemaphoreType.DMA,
            pltpu.SemaphoreType.DMA,
            pltpu.SemaphoreType.DMA,
            pltpu.SemaphoreType.DMA,
            pltpu.SemaphoreType.DMA,
        ],
        compiler_params=pltpu.CompilerParams(use_tc_tiling_on_sc=False),
    )
    def gather_kernel(table_hbm, idx_hbm, out_hbm, idx_v, rows_v,
                      sem_i0, sem_i1, sem_g, sem_o0, sem_o1):
        wid = lax.axis_index("s") * _NUM_CORES + lax.axis_index("c")
        base = wid * b_per_w
        sem_i = (sem_i0, sem_i1)
        sem_o = (sem_o0, sem_o1)

        for buf in range(2):
            pltpu.async_copy(
                idx_hbm.at[pl.ds(base + buf * chunk, chunk)],
                idx_v.at[buf], sem_i[buf])

        def body(go, carry):
            for buf in range(2):
                g = go * 2 + buf
                b0 = base + g * chunk
                pltpu.make_async_copy(
                    idx_hbm.at[pl.ds(0, chunk)], idx_v.at[buf],
                    sem_i[buf]).wait()
                @pl.when(go >= 1)
                def _():
                    pltpu.make_async_copy(
                        rows_v.at[buf], out_hbm.at[pl.ds(0, chunk)],
                        sem_o[buf]).wait()
                pltpu.async_copy(
                    table_hbm.at[idx_v.at[buf]], rows_v.at[buf], sem_g
                ).wait()
                @pl.when(go < n_outer - 1)
                def _():
                    pltpu.async_copy(
                        idx_hbm.at[pl.ds(b0 + 2 * chunk, chunk)],
                        idx_v.at[buf], sem_i[buf])
                pltpu.async_copy(
                    rows_v.at[buf], out_hbm.at[pl.ds(b0, chunk)], sem_o[buf])
            return carry

        lax.fori_loop(0, n_outer, body, 0)
        for buf in range(2):
            pltpu.make_async_copy(
                rows_v.at[buf], out_hbm.at[pl.ds(0, chunk)], sem_o[buf]).wait()

    return gather_kernel


def kernel(input_ids, embedding, W, b):
    batch, seq = input_ids.shape
    vocab, hidden = embedding.shape
    num_tokens = batch * seq
    table = _transform_table(embedding, W, b)
    flat_ids = input_ids.reshape(num_tokens).astype(jnp.int32)
    out = _make_gather(vocab, hidden, num_tokens, 512)(table, flat_ids)
    return out.reshape(batch, seq, hidden)
